# baseline (speedup 1.0000x reference)
import jax
import jax.numpy as jnp
from jax import lax
from jax.experimental import pallas as pl
from jax.experimental.pallas import tpu as pltpu
from jax.experimental.pallas import tpu_sc as plsc

N = 10000
NPAD = 10240
NC = 2
NS = 16
NW = NC * NS
CHUNK = 128
KCH = 80
KCH2 = 160
EPAD = NW * KCH * CHUNK
RPT = NPAD // NS
ZR = 128
BN = 1000

_SC_PARAMS = pltpu.CompilerParams(use_tc_tiling_on_sc=False)


def _zero_fill(ref, rows, width):
    z = jnp.zeros((16,), jnp.float32)

    def row(i, _):
        for k in range(width // 16):
            ref[i, pl.ds(k * 16, 16)] = z
        return 0

    lax.fori_loop(0, rows, row, 0)


def _make_agg(FH):
    mesh = plsc.VectorSubcoreMesh(core_axis_name="c", subcore_axis_name="s")

    def body(x_hbm, src_hbm, dst_hbm, out_hbm,
             src_v, dst_v, bufs, zero_v, acc_sh,
             gs0, gs1, gs2, gs3, ss0, ss1, ss2, ss3):
        gsems = [gs0, gs1, gs2, gs3]
        ssems = [ss0, ss1, ss2, ss3]
        c = lax.axis_index("c")
        s = lax.axis_index("s")
        icp0 = pltpu.async_copy(
            src_hbm.at[pl.ds((c * NS + s) * KCH2, KCH2)], src_v, gs0)
        icp1 = pltpu.async_copy(dst_hbm.at[pl.ds(s * KCH2, KCH2)], dst_v, gs1)
        _zero_fill(zero_v, ZR, FH)
        for i in range(RPT // ZR):
            pltpu.sync_copy(zero_v, acc_sh.at[pl.ds(s * RPT + i * ZR, ZR)])
        icp0.wait()
        icp1.wait()
        plsc.subcore_barrier()

        def gather(b, j):
            pltpu.async_copy(x_hbm.at[src_v.at[j]], bufs.at[b], gsems[b])

        def wait_gather(b):
            pltpu.make_async_copy(x_hbm.at[src_v.at[0]], bufs.at[b],
                                  gsems[b]).wait()

        def scatter(b, j):
            pltpu.async_copy(bufs.at[b], acc_sh.at[dst_v.at[j]], ssems[b],
                             add=True)

        def wait_scatter(b):
            pltpu.make_async_copy(bufs.at[b], acc_sh.at[dst_v.at[0]],
                                  ssems[b]).wait()

        gather(0, 0)
        gather(1, 1)
        for j in range(4):
            wait_gather(j)
            scatter(j, j)
            if j < 2:
                gather(j + 2, j + 2)
            else:
                wait_scatter(j - 2)
                gather(j - 2, j + 2)

        def step(g, _):
            for b in range(4):
                j = 4 * g + b
                wait_gather(b)
                scatter(b, j)
                b2 = (b + 2) % 4
                wait_scatter(b2)
                gather(b2, jnp.minimum(j + 2, KCH2 - 1))
            return 0

        lax.fori_loop(1, KCH2 // 4, step, 0)
        wait_gather(0)
        wait_gather(1)
        wait_scatter(2)
        wait_scatter(3)
        plsc.subcore_barrier()
        pltpu.sync_copy(acc_sh.at[pl.ds(s * RPT, RPT)],
                        out_hbm.at[pl.ds(c * NPAD + s * RPT, RPT)])

    return pl.kernel(
        body,
        out_type=jax.ShapeDtypeStruct((NC * NPAD, FH), jnp.float32),
        mesh=mesh,
        scratch_types=[
            pltpu.VMEM((KCH2, CHUNK), jnp.int32),
            pltpu.VMEM((KCH2, CHUNK), jnp.int32),
            pltpu.VMEM((4, CHUNK, FH), jnp.float32),
            pltpu.VMEM((ZR, FH), jnp.float32),
            pltpu.VMEM_SHARED((NPAD, FH), jnp.float32),
        ] + [pltpu.SemaphoreType.DMA] * 8,
        compiler_params=_SC_PARAMS,
    )


def _make_cnt():
    mesh = plsc.VectorSubcoreMesh(core_axis_name="c", subcore_axis_name="s")

    def body(dst_hbm, out_hbm, dst_v, ones_v, zero_v, cbuf, cnt_sh):
        c = lax.axis_index("c")
        s = lax.axis_index("s")
        wid = s * NC + c
        pltpu.sync_copy(dst_hbm.at[pl.ds(wid * KCH, KCH)], dst_v)
        one = jnp.ones((16,), jnp.float32)

        def row(i, _):
            ones_v[i, pl.ds(0, 16)] = one
            return 0

        lax.fori_loop(0, CHUNK, row, 0)
        _zero_fill(zero_v, RPT, 16)
        pltpu.sync_copy(zero_v, cnt_sh.at[pl.ds(s * RPT, RPT)])
        plsc.subcore_barrier()

        def step(j, _):
            pltpu.sync_copy(ones_v, cnt_sh.at[dst_v.at[j]], add=True)
            return 0

        lax.fori_loop(0, KCH, step, 0)
        plsc.subcore_barrier()
        pltpu.sync_copy(cnt_sh.at[pl.ds(s * RPT, RPT)], cbuf)
        pltpu.sync_copy(cbuf, out_hbm.at[pl.ds(c * NPAD + s * RPT, RPT)])

    return pl.kernel(
        body,
        out_type=jax.ShapeDtypeStruct((NC * NPAD, 16), jnp.float32),
        mesh=mesh,
        scratch_types=[
            pltpu.VMEM((KCH, CHUNK), jnp.int32),
            pltpu.VMEM((CHUNK, 16), jnp.float32),
            pltpu.VMEM((RPT, 16), jnp.float32),
            pltpu.VMEM((RPT, 16), jnp.float32),
            pltpu.VMEM_SHARED((NPAD, 16), jnp.float32),
        ],
        compiler_params=_SC_PARAMS,
    )


_agg64 = _make_agg(64)
_agg32 = _make_agg(32)
_cnt = _make_cnt()


def _split_cols(h, FW):
    FH = FW // 2
    return h.reshape(N, 2, FH).transpose(1, 0, 2).reshape(2 * N, FH)


def _acc_specs(FH):
    return [
        pl.BlockSpec((NC, BN, FH), lambda i: (0, i, 0)),
        pl.BlockSpec((NC, BN, 16), lambda i: (0, i, 0)),
    ]


def _row_spec(FW):
    return pl.BlockSpec((BN, FW), lambda i: (i, 0))


def _full_spec(a, b):
    return pl.BlockSpec((a, b), lambda i: (0, 0))


def _agg_of(acc_ref, cnt_ref):
    acc = jnp.concatenate([acc_ref[0], acc_ref[1]], axis=1)
    cnt = cnt_ref[0, :, 0:1] + cnt_ref[1, :, 0:1]
    return acc / jnp.maximum(cnt, 1.0), cnt


def _l1_body(acc_ref, cnt_ref, x_ref, wl_ref, wr_ref, b_ref, o_ref, os_ref):
    acc = jnp.concatenate([acc_ref[0], acc_ref[1]], axis=1)
    cnt = cnt_ref[0, :, 0:1] + cnt_ref[1, :, 0:1]
    agg = (acc * 0.001 - 0.5 * cnt) / jnp.maximum(cnt, 1.0)
    xb = x_ref[...] * 0.001 - 0.5
    h = jnp.dot(agg, wl_ref[...], preferred_element_type=jnp.float32)
    h = h + jnp.dot(xb, wr_ref[...], preferred_element_type=jnp.float32)
    h = jnp.sin(h + b_ref[...])
    o_ref[...] = h
    os_ref[0] = h[:, :64]
    os_ref[1] = h[:, 64:]


def _l2_body(acc_ref, cnt_ref, h1_ref, wl_ref, wr_ref, b_ref, eps_ref,
             mean_ref, logv_ref, z_ref, zs_ref):
    agg, _ = _agg_of(acc_ref, cnt_ref)
    h = jnp.dot(agg, wl_ref[...], preferred_element_type=jnp.float32)
    h = h + jnp.dot(h1_ref[...], wr_ref[...], preferred_element_type=jnp.float32)
    h = h + b_ref[...]
    mean = h[:, :64]
    logv = h[:, 64:]
    mean_ref[...] = mean
    logv_ref[...] = logv
    z = mean + jnp.exp(logv) * eps_ref[...]
    z_ref[...] = z
    zs_ref[0] = z[:, :32]
    zs_ref[1] = z[:, 32:]


def _l3_body(acc_ref, cnt_ref, z_ref, wl_ref, wr_ref, b_ref, o_ref, os_ref):
    agg, _ = _agg_of(acc_ref, cnt_ref)
    h = jnp.dot(agg, wl_ref[...], preferred_element_type=jnp.float32)
    h = h + jnp.dot(z_ref[...], wr_ref[...], preferred_element_type=jnp.float32)
    h = jnp.maximum(h + b_ref[...], 0.0)
    o_ref[...] = h
    os_ref[0] = h[:, :32]
    os_ref[1] = h[:, 32:]


def _l4_body(acc_ref, cnt_ref, h3_ref, wl_ref, wr_ref, b_ref,
             wlin_ref, blin_ref, o_ref):
    agg, _ = _agg_of(acc_ref, cnt_ref)
    h = jnp.dot(agg, wl_ref[...], preferred_element_type=jnp.float32)
    h = h + jnp.dot(h3_ref[...], wr_ref[...], preferred_element_type=jnp.float32)
    h4 = jnp.maximum(h + b_ref[...], 0.0)
    o = jnp.dot(h4, wlin_ref[...], preferred_element_type=jnp.float32)
    o_ref[...] = jax.nn.sigmoid(o + blin_ref[...]) * 1000.0


_GRID = (N // BN,)


def _split_out_spec(FH):
    return pl.BlockSpec((2, BN, FH), lambda i: (0, i, 0))


def _tc1(acc, cnt, xp, wl, wr, b):
    return pl.pallas_call(
        _l1_body,
        grid=_GRID,
        in_specs=_acc_specs(64) + [_row_spec(128), _full_spec(128, 128),
                                   _full_spec(128, 128), _full_spec(1, 128)],
        out_specs=[_row_spec(128), _split_out_spec(64)],
        out_shape=[jax.ShapeDtypeStruct((N, 128), jnp.float32),
                   jax.ShapeDtypeStruct((2, N, 64), jnp.float32)],
    )(acc, cnt, xp, wl, wr, b)


def _tc2(acc, cnt, h1, wl, wr, b, eps):
    s64 = jax.ShapeDtypeStruct((N, 64), jnp.float32)
    return pl.pallas_call(
        _l2_body,
        grid=_GRID,
        in_specs=_acc_specs(64) + [_row_spec(128), _full_spec(128, 128),
                                   _full_spec(128, 128), _full_spec(1, 128),
                                   _row_spec(64)],
        out_specs=[_row_spec(64), _row_spec(64), _row_spec(64),
                   _split_out_spec(32)],
        out_shape=[s64, s64, s64,
                   jax.ShapeDtypeStruct((2, N, 32), jnp.float32)],
    )(acc, cnt, h1, wl, wr, b, eps)


def _tc3(acc, cnt, z, wl, wr, b):
    return pl.pallas_call(
        _l3_body,
        grid=_GRID,
        in_specs=_acc_specs(32) + [_row_spec(64), _full_spec(64, 64),
                                   _full_spec(64, 64), _full_spec(1, 64)],
        out_specs=[_row_spec(64), _split_out_spec(32)],
        out_shape=[jax.ShapeDtypeStruct((N, 64), jnp.float32),
                   jax.ShapeDtypeStruct((2, N, 32), jnp.float32)],
    )(acc, cnt, z, wl, wr, b)


def _tc4(acc, cnt, h3, wl, wr, b, wlin, blin):
    return pl.pallas_call(
        _l4_body,
        grid=_GRID,
        in_specs=_acc_specs(32) + [_row_spec(64), _full_spec(64, 64),
                                   _full_spec(64, 64), _full_spec(1, 64),
                                   _full_spec(64, 128), _full_spec(1, 128)],
        out_specs=_row_spec(128),
        out_shape=jax.ShapeDtypeStruct((N, 128), jnp.float32),
    )(acc, cnt, h3, wl, wr, b, wlin, blin)


def kernel(x, edge_index, Wl1, Wr1, b1, Wl2, Wr2, b2, Wl3, Wr3, b3,
           Wl4, Wr4, b4, Wlin, blin, eps):
    E = edge_index.shape[1]
    pad = EPAD - E
    ar = jnp.arange(pad, dtype=jnp.int32)
    pad_src = (ar * 131) % N
    pad_dst = N + ar % (NPAD - N)
    src = jnp.concatenate([edge_index[0], pad_src])
    dst = jnp.concatenate([edge_index[1], pad_dst])
    src2 = jnp.stack([src, src + N]).reshape(NC * NS * KCH2, CHUNK)
    dstp = dst.reshape(NS * KCH2, CHUNK)
    cnt = _cnt(dstp).reshape(NC, NPAD, 16)
    acc1 = _agg64(_split_cols(x, 128), src2, dstp).reshape(NC, NPAD, 64)
    h1, h1s = _tc1(acc1, cnt, x, Wl1, Wr1, b1.reshape(1, -1))
    acc2 = _agg64(h1s.reshape(2 * N, 64), src2, dstp).reshape(NC, NPAD, 64)
    mean, logv, z, zs = _tc2(acc2, cnt, h1, Wl2, Wr2, b2.reshape(1, -1), eps)
    acc3 = _agg32(zs.reshape(2 * N, 32), src2, dstp).reshape(NC, NPAD, 32)
    h3, h3s = _tc3(acc3, cnt, z, Wl3, Wr3, b3.reshape(1, -1))
    acc4 = _agg32(h3s.reshape(2 * N, 32), src2, dstp).reshape(NC, NPAD, 32)
    out = _tc4(acc4, cnt, h3, Wl4, Wr4, b4.reshape(1, -1), Wlin,
               blin.reshape(1, -1))
    return out, mean, logv


# speedup vs baseline: 1.0094x; 1.0094x over previous
import jax
import jax.numpy as jnp
from jax import lax
from jax.experimental import pallas as pl
from jax.experimental.pallas import tpu as pltpu
from jax.experimental.pallas import tpu_sc as plsc

N = 10000
NPAD = 10240
NC = 2
NS = 16
NW = NC * NS
CHUNK = 128
KCH = 80
KCH2 = 160
EPAD = NW * KCH * CHUNK
RPT = NPAD // NS
ZR = 128
BN = 2000

_SC_PARAMS = pltpu.CompilerParams(use_tc_tiling_on_sc=False)


def _zero_fill(ref, rows, width):
    z = jnp.zeros((16,), jnp.float32)

    def row(i, _):
        for k in range(width // 16):
            ref[i, pl.ds(k * 16, 16)] = z
        return 0

    lax.fori_loop(0, rows, row, 0)


def _make_agg(FH):
    mesh = plsc.VectorSubcoreMesh(core_axis_name="c", subcore_axis_name="s")

    def body(x_hbm, src_hbm, dst_hbm, out_hbm,
             src_v, dst_v, bufs, zero_v, acc_sh,
             gs0, gs1, gs2, gs3, ss0, ss1, ss2, ss3):
        gsems = [gs0, gs1, gs2, gs3]
        ssems = [ss0, ss1, ss2, ss3]
        c = lax.axis_index("c")
        s = lax.axis_index("s")
        icp0 = pltpu.async_copy(
            src_hbm.at[pl.ds((c * NS + s) * KCH2, KCH2)], src_v, gs0)
        icp1 = pltpu.async_copy(dst_hbm.at[pl.ds(s * KCH2, KCH2)], dst_v, gs1)
        _zero_fill(zero_v, ZR, FH)
        for i in range(RPT // ZR):
            pltpu.sync_copy(zero_v, acc_sh.at[pl.ds(s * RPT + i * ZR, ZR)])
        icp0.wait()
        icp1.wait()
        plsc.subcore_barrier()

        def gather(b, j):
            pltpu.async_copy(x_hbm.at[src_v.at[j]], bufs.at[b], gsems[b])

        def wait_gather(b):
            pltpu.make_async_copy(x_hbm.at[src_v.at[0]], bufs.at[b],
                                  gsems[b]).wait()

        def scatter(b, j):
            pltpu.async_copy(bufs.at[b], acc_sh.at[dst_v.at[j]], ssems[b],
                             add=True)

        def wait_scatter(b):
            pltpu.make_async_copy(bufs.at[b], acc_sh.at[dst_v.at[0]],
                                  ssems[b]).wait()

        gather(0, 0)
        gather(1, 1)
        for j in range(4):
            wait_gather(j)
            scatter(j, j)
            if j < 2:
                gather(j + 2, j + 2)
            else:
                wait_scatter(j - 2)
                gather(j - 2, j + 2)

        def step(g, _):
            for b in range(4):
                j = 4 * g + b
                wait_gather(b)
                scatter(b, j)
                b2 = (b + 2) % 4
                wait_scatter(b2)
                gather(b2, jnp.minimum(j + 2, KCH2 - 1))
            return 0

        lax.fori_loop(1, KCH2 // 4, step, 0)
        wait_gather(0)
        wait_gather(1)
        wait_scatter(2)
        wait_scatter(3)
        plsc.subcore_barrier()
        pltpu.sync_copy(acc_sh.at[pl.ds(s * RPT, RPT)],
                        out_hbm.at[pl.ds(c * NPAD + s * RPT, RPT)])

    return pl.kernel(
        body,
        out_type=jax.ShapeDtypeStruct((NC * NPAD, FH), jnp.float32),
        mesh=mesh,
        scratch_types=[
            pltpu.VMEM((KCH2, CHUNK), jnp.int32),
            pltpu.VMEM((KCH2, CHUNK), jnp.int32),
            pltpu.VMEM((4, CHUNK, FH), jnp.float32),
            pltpu.VMEM((ZR, FH), jnp.float32),
            pltpu.VMEM_SHARED((NPAD, FH), jnp.float32),
        ] + [pltpu.SemaphoreType.DMA] * 8,
        compiler_params=_SC_PARAMS,
    )


def _make_cnt():
    mesh = plsc.VectorSubcoreMesh(core_axis_name="c", subcore_axis_name="s")

    def body(dst_hbm, out_hbm, dst_v, ones_v, zero_v, cbuf, cnt_sh):
        c = lax.axis_index("c")
        s = lax.axis_index("s")
        wid = s * NC + c
        pltpu.sync_copy(dst_hbm.at[pl.ds(wid * KCH, KCH)], dst_v)
        one = jnp.ones((16,), jnp.float32)

        def row(i, _):
            ones_v[i, pl.ds(0, 16)] = one
            return 0

        lax.fori_loop(0, CHUNK, row, 0)
        _zero_fill(zero_v, RPT, 16)
        pltpu.sync_copy(zero_v, cnt_sh.at[pl.ds(s * RPT, RPT)])
        plsc.subcore_barrier()

        def step(j, _):
            pltpu.sync_copy(ones_v, cnt_sh.at[dst_v.at[j]], add=True)
            return 0

        lax.fori_loop(0, KCH, step, 0)
        plsc.subcore_barrier()
        pltpu.sync_copy(cnt_sh.at[pl.ds(s * RPT, RPT)], cbuf)
        pltpu.sync_copy(cbuf, out_hbm.at[pl.ds(c * NPAD + s * RPT, RPT)])

    return pl.kernel(
        body,
        out_type=jax.ShapeDtypeStruct((NC * NPAD, 16), jnp.float32),
        mesh=mesh,
        scratch_types=[
            pltpu.VMEM((KCH, CHUNK), jnp.int32),
            pltpu.VMEM((CHUNK, 16), jnp.float32),
            pltpu.VMEM((RPT, 16), jnp.float32),
            pltpu.VMEM((RPT, 16), jnp.float32),
            pltpu.VMEM_SHARED((NPAD, 16), jnp.float32),
        ],
        compiler_params=_SC_PARAMS,
    )


_agg64 = _make_agg(64)
_agg32 = _make_agg(32)
_cnt = _make_cnt()


def _split_cols(h, FW):
    FH = FW // 2
    return h.reshape(N, 2, FH).transpose(1, 0, 2).reshape(2 * N, FH)


def _acc_specs(FH):
    return [
        pl.BlockSpec((NC, BN, FH), lambda i: (0, i, 0)),
        pl.BlockSpec((NC, BN, 16), lambda i: (0, i, 0)),
    ]


def _row_spec(FW):
    return pl.BlockSpec((BN, FW), lambda i: (i, 0))


def _full_spec(a, b):
    return pl.BlockSpec((a, b), lambda i: (0, 0))


def _agg_of(acc_ref, cnt_ref):
    acc = jnp.concatenate([acc_ref[0], acc_ref[1]], axis=1)
    cnt = cnt_ref[0, :, 0:1] + cnt_ref[1, :, 0:1]
    return acc / jnp.maximum(cnt, 1.0), cnt


def _l1_body(acc_ref, cnt_ref, x_ref, wl_ref, wr_ref, b_ref, o_ref, os_ref):
    acc = jnp.concatenate([acc_ref[0], acc_ref[1]], axis=1)
    cnt = cnt_ref[0, :, 0:1] + cnt_ref[1, :, 0:1]
    agg = (acc * 0.001 - 0.5 * cnt) / jnp.maximum(cnt, 1.0)
    xb = x_ref[...] * 0.001 - 0.5
    h = jnp.dot(agg, wl_ref[...], preferred_element_type=jnp.float32)
    h = h + jnp.dot(xb, wr_ref[...], preferred_element_type=jnp.float32)
    h = jnp.sin(h + b_ref[...])
    o_ref[...] = h
    os_ref[0] = h[:, :64]
    os_ref[1] = h[:, 64:]


def _l2_body(acc_ref, cnt_ref, h1_ref, wl_ref, wr_ref, b_ref, eps_ref,
             mean_ref, logv_ref, z_ref, zs_ref):
    agg, _ = _agg_of(acc_ref, cnt_ref)
    h = jnp.dot(agg, wl_ref[...], preferred_element_type=jnp.float32)
    h = h + jnp.dot(h1_ref[...], wr_ref[...], preferred_element_type=jnp.float32)
    h = h + b_ref[...]
    mean = h[:, :64]
    logv = h[:, 64:]
    mean_ref[...] = mean
    logv_ref[...] = logv
    z = mean + jnp.exp(logv) * eps_ref[...]
    z_ref[...] = z
    zs_ref[0] = z[:, :32]
    zs_ref[1] = z[:, 32:]


def _l3_body(acc_ref, cnt_ref, z_ref, wl_ref, wr_ref, b_ref, o_ref, os_ref):
    agg, _ = _agg_of(acc_ref, cnt_ref)
    h = jnp.dot(agg, wl_ref[...], preferred_element_type=jnp.float32)
    h = h + jnp.dot(z_ref[...], wr_ref[...], preferred_element_type=jnp.float32)
    h = jnp.maximum(h + b_ref[...], 0.0)
    o_ref[...] = h
    os_ref[0] = h[:, :32]
    os_ref[1] = h[:, 32:]


def _l4_body(acc_ref, cnt_ref, h3_ref, wl_ref, wr_ref, b_ref,
             wlin_ref, blin_ref, o_ref):
    agg, _ = _agg_of(acc_ref, cnt_ref)
    h = jnp.dot(agg, wl_ref[...], preferred_element_type=jnp.float32)
    h = h + jnp.dot(h3_ref[...], wr_ref[...], preferred_element_type=jnp.float32)
    h4 = jnp.maximum(h + b_ref[...], 0.0)
    o = jnp.dot(h4, wlin_ref[...], preferred_element_type=jnp.float32)
    o_ref[...] = jax.nn.sigmoid(o + blin_ref[...]) * 1000.0


_GRID = (N // BN,)


def _split_out_spec(FH):
    return pl.BlockSpec((2, BN, FH), lambda i: (0, i, 0))


def _tc1(acc, cnt, xp, wl, wr, b):
    return pl.pallas_call(
        _l1_body,
        grid=_GRID,
        in_specs=_acc_specs(64) + [_row_spec(128), _full_spec(128, 128),
                                   _full_spec(128, 128), _full_spec(1, 128)],
        out_specs=[_row_spec(128), _split_out_spec(64)],
        out_shape=[jax.ShapeDtypeStruct((N, 128), jnp.float32),
                   jax.ShapeDtypeStruct((2, N, 64), jnp.float32)],
    )(acc, cnt, xp, wl, wr, b)


def _tc2(acc, cnt, h1, wl, wr, b, eps):
    s64 = jax.ShapeDtypeStruct((N, 64), jnp.float32)
    return pl.pallas_call(
        _l2_body,
        grid=_GRID,
        in_specs=_acc_specs(64) + [_row_spec(128), _full_spec(128, 128),
                                   _full_spec(128, 128), _full_spec(1, 128),
                                   _row_spec(64)],
        out_specs=[_row_spec(64), _row_spec(64), _row_spec(64),
                   _split_out_spec(32)],
        out_shape=[s64, s64, s64,
                   jax.ShapeDtypeStruct((2, N, 32), jnp.float32)],
    )(acc, cnt, h1, wl, wr, b, eps)


def _tc3(acc, cnt, z, wl, wr, b):
    return pl.pallas_call(
        _l3_body,
        grid=_GRID,
        in_specs=_acc_specs(32) + [_row_spec(64), _full_spec(64, 64),
                                   _full_spec(64, 64), _full_spec(1, 64)],
        out_specs=[_row_spec(64), _split_out_spec(32)],
        out_shape=[jax.ShapeDtypeStruct((N, 64), jnp.float32),
                   jax.ShapeDtypeStruct((2, N, 32), jnp.float32)],
    )(acc, cnt, z, wl, wr, b)


def _tc4(acc, cnt, h3, wl, wr, b, wlin, blin):
    return pl.pallas_call(
        _l4_body,
        grid=_GRID,
        in_specs=_acc_specs(32) + [_row_spec(64), _full_spec(64, 64),
                                   _full_spec(64, 64), _full_spec(1, 64),
                                   _full_spec(64, 128), _full_spec(1, 128)],
        out_specs=_row_spec(128),
        out_shape=jax.ShapeDtypeStruct((N, 128), jnp.float32),
    )(acc, cnt, h3, wl, wr, b, wlin, blin)


def kernel(x, edge_index, Wl1, Wr1, b1, Wl2, Wr2, b2, Wl3, Wr3, b3,
           Wl4, Wr4, b4, Wlin, blin, eps):
    E = edge_index.shape[1]
    pad = EPAD - E
    ar = jnp.arange(pad, dtype=jnp.int32)
    pad_src = (ar * 131) % N
    pad_dst = N + ar % (NPAD - N)
    src = jnp.concatenate([edge_index[0], pad_src])
    dst = jnp.concatenate([edge_index[1], pad_dst])
    src2 = jnp.stack([src, src + N]).reshape(NC * NS * KCH2, CHUNK)
    dstp = dst.reshape(NS * KCH2, CHUNK)
    cnt = _cnt(dstp).reshape(NC, NPAD, 16)
    acc1 = _agg64(_split_cols(x, 128), src2, dstp).reshape(NC, NPAD, 64)
    h1, h1s = _tc1(acc1, cnt, x, Wl1, Wr1, b1.reshape(1, -1))
    acc2 = _agg64(h1s.reshape(2 * N, 64), src2, dstp).reshape(NC, NPAD, 64)
    mean, logv, z, zs = _tc2(acc2, cnt, h1, Wl2, Wr2, b2.reshape(1, -1), eps)
    acc3 = _agg32(zs.reshape(2 * N, 32), src2, dstp).reshape(NC, NPAD, 32)
    h3, h3s = _tc3(acc3, cnt, z, Wl3, Wr3, b3.reshape(1, -1))
    acc4 = _agg32(h3s.reshape(2 * N, 32), src2, dstp).reshape(NC, NPAD, 32)
    out = _tc4(acc4, cnt, h3, Wl4, Wr4, b4.reshape(1, -1), Wlin,
               blin.reshape(1, -1))
    return out, mean, logv
